# baseline (speedup 1.0000x reference)
import functools

import jax
import jax.numpy as jnp
from jax import lax
from jax.experimental import pallas as pl
from jax.experimental.pallas import tpu as pltpu
from jax.experimental.pallas import tpu_sc as plsc

BATCH = 16384
FACTORS = 32
NUM_WORKERS = 32
B_PER_W = BATCH // NUM_WORKERS
LANES = 16
GROUPS = B_PER_W // LANES


def _sc_body(user_hbm, item_hbm, utab_hbm, itab_hbm, out_hbm,
             uidx_v, iidx_v, urows_v, vrows_v, outv, sem_u, sem_v):
    wid = lax.axis_index("s") * 2 + lax.axis_index("c")
    base = wid * B_PER_W

    pltpu.sync_copy(user_hbm.at[pl.ds(base, B_PER_W)], uidx_v)
    pltpu.sync_copy(item_hbm.at[pl.ds(base, B_PER_W)], iidx_v)

    cu = pltpu.async_copy(utab_hbm.at[uidx_v], urows_v, sem_u)
    cv = pltpu.async_copy(itab_hbm.at[iidx_v], vrows_v, sem_v)
    cu.wait()
    cv.wait()

    lane = lax.iota(jnp.int32, 16)

    def group_body(g, carry):
        acc = jnp.zeros((16,), jnp.float32)
        rows = lane + g * LANES
        for f in range(FACTORS):
            cols = jnp.full((16,), f, jnp.int32)
            u = plsc.load_gather(urows_v, [rows, cols])
            v = plsc.load_gather(vrows_v, [rows, cols])
            acc = acc + u * v
        outv[pl.ds(pl.multiple_of(g * LANES, LANES), LANES)] = acc
        return carry

    lax.fori_loop(0, GROUPS, group_body, 0)

    pltpu.sync_copy(outv, out_hbm.at[pl.ds(base, B_PER_W)])


def kernel(user, item, user_table, item_table):
    mesh = plsc.VectorSubcoreMesh(core_axis_name="c", subcore_axis_name="s")
    k = functools.partial(
        pl.kernel,
        out_type=jax.ShapeDtypeStruct((BATCH,), jnp.float32),
        mesh=mesh,
        compiler_params=pltpu.CompilerParams(needs_layout_passes=False, use_tc_tiling_on_sc=False),
        scratch_types=[
            pltpu.VMEM((B_PER_W,), jnp.int32),
            pltpu.VMEM((B_PER_W,), jnp.int32),
            pltpu.VMEM((B_PER_W, FACTORS), jnp.float32),
            pltpu.VMEM((B_PER_W, FACTORS), jnp.float32),
            pltpu.VMEM((B_PER_W,), jnp.float32),
            pltpu.SemaphoreType.DMA,
            pltpu.SemaphoreType.DMA,
        ],
    )(_sc_body)
    return k(user, item, user_table, item_table)
